# Initial kernel scaffold; baseline (speedup 1.0000x reference)
import jax
import jax.numpy as jnp
from jax import lax
from jax.experimental import pallas as pl
from jax.experimental.pallas import tpu as pltpu
from jax.experimental.pallas import tpu_sc as plsc

_B, _T, _DIM = 16, 576, 768
_G, _N, _VD = 2, 1024, 256
_ROWS = _B * _T
_TILE = 576
_NSTEPS = _ROWS // _TILE
_TEMP = 2.0

_NW = 32
_PER_W = _ROWS * _G // _NW
_CH = 96
_NCH = _PER_W // _CH


def _tc_body(x_ref, wt_ref, b_ref, idx_ref, cperp_ref, pperp_ref, pacc, hacc):
    step = pl.program_id(0)

    @pl.when(step == 0)
    def _init():
        pacc[...] = jnp.zeros_like(pacc)
        hacc[...] = jnp.zeros_like(hacc)

    logits = (
        jnp.dot(x_ref[...], wt_ref[...], preferred_element_type=jnp.float32)
        + b_ref[...]
    )
    iota = lax.broadcasted_iota(jnp.int32, (_TILE, _N), 1)
    for g in range(_G):
        lg = logits[:, g * _N:(g + 1) * _N]
        m = jnp.max(lg, axis=1, keepdims=True)
        idx = jnp.min(jnp.where(lg >= m, iota, _N), axis=1, keepdims=True)
        e = jnp.exp(lg - m)
        p = e / jnp.sum(e, axis=1, keepdims=True)
        pacc[g:g + 1, :] += jnp.sum(p, axis=0, keepdims=True)
        onehot = (iota == idx).astype(jnp.float32)
        hacc[g:g + 1, :] += jnp.sum(onehot, axis=0, keepdims=True)
        idx_ref[:, g:g + 1] = idx + g * _N

    @pl.when(step == _NSTEPS - 1)
    def _finish():
        inv = 1.0 / _ROWS
        hp = hacc[...] * inv
        cperp_ref[0, 0] = jnp.sum(
            jnp.exp(-jnp.sum(hp * jnp.log(hp + 1e-7), axis=1, keepdims=True)))
        ap = pacc[...] * inv
        pperp_ref[0, 0] = jnp.sum(
            jnp.exp(-jnp.sum(ap * jnp.log(ap + 1e-7), axis=1, keepdims=True)))


def _tc_call(xf, wt, b2):
    return pl.pallas_call(
        _tc_body,
        grid=(_NSTEPS,),
        in_specs=[
            pl.BlockSpec((_TILE, _DIM), lambda i: (i, 0)),
            pl.BlockSpec((_DIM, _G * _N), lambda i: (0, 0)),
            pl.BlockSpec((1, _G * _N), lambda i: (0, 0)),
        ],
        out_specs=[
            pl.BlockSpec((_TILE, _G), lambda i: (i, 0)),
            pl.BlockSpec((1, 1), lambda i: (0, 0)),
            pl.BlockSpec((1, 1), lambda i: (0, 0)),
        ],
        out_shape=[
            jax.ShapeDtypeStruct((_ROWS, _G), jnp.int32),
            jax.ShapeDtypeStruct((1, 1), jnp.float32),
            jax.ShapeDtypeStruct((1, 1), jnp.float32),
        ],
        scratch_shapes=[
            pltpu.VMEM((_G, _N), jnp.float32),
            pltpu.VMEM((_G, _N), jnp.float32),
        ],
    )(xf, wt, b2)


def _sc_body(table_hbm, idx_hbm, out_hbm, idx_v, rows_v, sem):
    c = lax.axis_index("c")
    s = lax.axis_index("s")
    wid = s * 2 + c
    pltpu.sync_copy(idx_hbm.at[wid], idx_v)
    for j in range(_NCH):
        pltpu.async_copy(table_hbm.at[idx_v.at[j]], rows_v, sem).wait()
        pltpu.sync_copy(rows_v, out_hbm.at[pl.ds(wid * _PER_W + j * _CH, _CH)])


def _sc_gather(table, idx3):
    mesh = plsc.VectorSubcoreMesh(core_axis_name="c", subcore_axis_name="s")
    return pl.kernel(
        _sc_body,
        out_type=jax.ShapeDtypeStruct((_ROWS * _G, _VD), jnp.float32),
        mesh=mesh,
        scratch_types=[
            pltpu.VMEM((_NCH, _CH), jnp.int32),
            pltpu.VMEM((_CH, _VD), jnp.float32),
            pltpu.SemaphoreType.DMA,
        ],
    )(table, idx3)


def kernel(x, W, b, codebook):
    xf = x.reshape(_ROWS, _DIM)
    wt = W.T
    b2 = b.reshape(1, _G * _N)
    idx, cperp, pperp = _tc_call(xf, wt, b2)
    idx3 = idx.reshape(_NW, _NCH, _CH)
    table = codebook.reshape(_G * _N, _VD)
    rows = _sc_gather(table, idx3)
    out = rows.reshape(_B, _T, _G * _VD)
    return out, cperp.reshape(()), pperp.reshape(())


# speedup vs baseline: 5.4559x; 5.4559x over previous
import jax
import jax.numpy as jnp
from jax import lax
from jax.experimental import pallas as pl
from jax.experimental.pallas import tpu as pltpu
from jax.experimental.pallas import tpu_sc as plsc

_B, _T, _DIM = 16, 576, 768
_G, _N, _VD = 2, 1024, 256
_ROWS = _B * _T
_TILE = 576
_NSTEPS = _ROWS // _TILE
_TEMP = 2.0

_NW = 32
_PER_W = _ROWS * _G // _NW
_CH = 96
_NCH = _PER_W // _CH


def _tc_body(x_ref, wt_ref, b_ref, idx_ref, cperp_ref, pperp_ref, pacc, hacc):
    step = pl.program_id(0)

    @pl.when(step == 0)
    def _init():
        pacc[...] = jnp.zeros_like(pacc)
        hacc[...] = jnp.zeros_like(hacc)

    logits = (
        jnp.dot(x_ref[...], wt_ref[...], preferred_element_type=jnp.float32)
        + b_ref[...]
    )
    iota = lax.broadcasted_iota(jnp.int32, (_TILE, _N), 1)
    for g in range(_G):
        lg = logits[:, g * _N:(g + 1) * _N]
        m = jnp.max(lg, axis=1, keepdims=True)
        idx = jnp.min(jnp.where(lg >= m, iota, _N), axis=1, keepdims=True)
        e = jnp.exp(lg - m)
        p = e / jnp.sum(e, axis=1, keepdims=True)
        pacc[g:g + 1, :] += jnp.sum(p, axis=0, keepdims=True)
        onehot = (iota == idx).astype(jnp.float32)
        hacc[g:g + 1, :] += jnp.sum(onehot, axis=0, keepdims=True)
        idx_ref[:, g:g + 1] = idx + g * _N

    @pl.when(step == _NSTEPS - 1)
    def _finish():
        inv = 1.0 / _ROWS
        hp = hacc[...] * inv
        ent_h = jnp.exp(-jnp.sum(hp * jnp.log(hp + 1e-7), axis=1, keepdims=True))
        cperp_ref[...] = jnp.sum(ent_h, axis=0, keepdims=True)
        ap = pacc[...] * inv
        ent_p = jnp.exp(-jnp.sum(ap * jnp.log(ap + 1e-7), axis=1, keepdims=True))
        pperp_ref[...] = jnp.sum(ent_p, axis=0, keepdims=True)


def _tc_call(xf, wt, b2):
    return pl.pallas_call(
        _tc_body,
        grid=(_NSTEPS,),
        in_specs=[
            pl.BlockSpec((_TILE, _DIM), lambda i: (i, 0)),
            pl.BlockSpec((_DIM, _G * _N), lambda i: (0, 0)),
            pl.BlockSpec((1, _G * _N), lambda i: (0, 0)),
        ],
        out_specs=[
            pl.BlockSpec((_TILE, _G), lambda i: (i, 0)),
            pl.BlockSpec((1, 1), lambda i: (0, 0)),
            pl.BlockSpec((1, 1), lambda i: (0, 0)),
        ],
        out_shape=[
            jax.ShapeDtypeStruct((_ROWS, _G), jnp.int32),
            jax.ShapeDtypeStruct((1, 1), jnp.float32),
            jax.ShapeDtypeStruct((1, 1), jnp.float32),
        ],
        scratch_shapes=[
            pltpu.VMEM((_G, _N), jnp.float32),
            pltpu.VMEM((_G, _N), jnp.float32),
        ],
    )(xf, wt, b2)


def _sc_body(table_hbm, idx_hbm, out_hbm, idx_v, rows_v, sem):
    c = lax.axis_index("c")
    s = lax.axis_index("s")
    wid = s * 2 + c
    pltpu.sync_copy(idx_hbm.at[wid], idx_v)
    for j in range(_NCH):
        pltpu.async_copy(table_hbm.at[idx_v.at[j]], rows_v, sem).wait()
        pltpu.sync_copy(rows_v, out_hbm.at[pl.ds(wid * _PER_W + j * _CH, _CH)])


def _sc_gather(table, idx3):
    mesh = plsc.VectorSubcoreMesh(core_axis_name="c", subcore_axis_name="s")
    return pl.kernel(
        _sc_body,
        out_type=jax.ShapeDtypeStruct((_ROWS * _G, _VD), jnp.float32),
        mesh=mesh,
        scratch_types=[
            pltpu.VMEM((_NCH, _CH), jnp.int32),
            pltpu.VMEM((_CH, _VD), jnp.float32),
            pltpu.SemaphoreType.DMA,
        ],
    )(table, idx3)


def kernel(x, W, b, codebook):
    xf = x.reshape(_ROWS, _DIM)
    wt = W.T
    b2 = b.reshape(1, _G * _N)
    idx, cperp, pperp = _tc_call(xf, wt, b2)
    idx3 = idx.reshape(_NW, _NCH, _CH)
    table = codebook.reshape(_G * _N, _VD)
    rows = _sc_gather(table, idx3)
    out = rows.reshape(_B, _T, _G * _VD)
    return out, cperp.reshape(()), pperp.reshape(())


# speedup vs baseline: 6.0537x; 1.1096x over previous
import jax
import jax.numpy as jnp
from jax import lax
from jax.experimental import pallas as pl
from jax.experimental.pallas import tpu as pltpu
from jax.experimental.pallas import tpu_sc as plsc

_B, _T, _DIM = 16, 576, 768
_G, _N, _VD = 2, 1024, 256
_ROWS = _B * _T
_TILE = 576
_NSTEPS = _ROWS // _TILE
_TEMP = 2.0

_NW = 32
_PER_W = _ROWS * _G // _NW
_CH = 96
_NCH = _PER_W // _CH


def _tc_body(x_ref, wt_ref, b_ref, idx_ref, cperp_ref, pperp_ref, pacc, hacc):
    step = pl.program_id(0)

    @pl.when(step == 0)
    def _init():
        pacc[...] = jnp.zeros_like(pacc)
        hacc[...] = jnp.zeros_like(hacc)

    logits = lax.dot_general(
        x_ref[...], wt_ref[...],
        (((1,), (1,)), ((), ())),
        preferred_element_type=jnp.float32,
    ) + b_ref[...]
    iota = lax.broadcasted_iota(jnp.int32, (_TILE, _N), 1)
    for g in range(_G):
        lg = logits[:, g * _N:(g + 1) * _N]
        m = jnp.max(lg, axis=1, keepdims=True)
        idx = jnp.min(jnp.where(lg >= m, iota, _N), axis=1, keepdims=True)
        e = jnp.exp(lg - m)
        r = 1.0 / jnp.sum(e, axis=1, keepdims=True)
        onehot = (iota == idx).astype(jnp.float32)
        pacc[g:g + 1, :] += lax.dot_general(
            r, e, (((0,), (0,)), ((), ())),
            preferred_element_type=jnp.float32)
        hacc[g:g + 1, :] += lax.dot_general(
            jnp.ones((_TILE, 1), jnp.float32), onehot,
            (((0,), (0,)), ((), ())),
            preferred_element_type=jnp.float32)
        idx_ref[:, g:g + 1] = idx + g * _N

    @pl.when(step == _NSTEPS - 1)
    def _finish():
        inv = 1.0 / _ROWS
        hp = hacc[...] * inv
        ent_h = jnp.exp(-jnp.sum(hp * jnp.log(hp + 1e-7), axis=1, keepdims=True))
        cperp_ref[...] = jnp.sum(ent_h, axis=0, keepdims=True)
        ap = pacc[...] * inv
        ent_p = jnp.exp(-jnp.sum(ap * jnp.log(ap + 1e-7), axis=1, keepdims=True))
        pperp_ref[...] = jnp.sum(ent_p, axis=0, keepdims=True)


def _tc_call(xf, wt, b2):
    return pl.pallas_call(
        _tc_body,
        grid=(_NSTEPS,),
        in_specs=[
            pl.BlockSpec((_TILE, _DIM), lambda i: (i, 0)),
            pl.BlockSpec((_G * _N, _DIM), lambda i: (0, 0)),
            pl.BlockSpec((1, _G * _N), lambda i: (0, 0)),
        ],
        out_specs=[
            pl.BlockSpec((_TILE, _G), lambda i: (i, 0)),
            pl.BlockSpec((1, 1), lambda i: (0, 0)),
            pl.BlockSpec((1, 1), lambda i: (0, 0)),
        ],
        out_shape=[
            jax.ShapeDtypeStruct((_ROWS, _G), jnp.int32),
            jax.ShapeDtypeStruct((1, 1), jnp.float32),
            jax.ShapeDtypeStruct((1, 1), jnp.float32),
        ],
        scratch_shapes=[
            pltpu.VMEM((_G, _N), jnp.float32),
            pltpu.VMEM((_G, _N), jnp.float32),
        ],
    )(xf, wt, b2)


def _sc_body(table_hbm, idx_hbm, out_hbm, idx_v, rows_v, sem):
    c = lax.axis_index("c")
    s = lax.axis_index("s")
    wid = s * 2 + c
    pltpu.sync_copy(idx_hbm.at[wid], idx_v)
    for j in range(_NCH):
        pltpu.async_copy(table_hbm.at[idx_v.at[j]], rows_v, sem).wait()
        pltpu.sync_copy(rows_v, out_hbm.at[pl.ds(wid * _PER_W + j * _CH, _CH)])


def _sc_gather(table, idx3):
    mesh = plsc.VectorSubcoreMesh(core_axis_name="c", subcore_axis_name="s")
    return pl.kernel(
        _sc_body,
        out_type=jax.ShapeDtypeStruct((_ROWS * _G, _VD), jnp.float32),
        mesh=mesh,
        scratch_types=[
            pltpu.VMEM((_NCH, _CH), jnp.int32),
            pltpu.VMEM((_CH, _VD), jnp.float32),
            pltpu.SemaphoreType.DMA,
        ],
    )(table, idx3)


def kernel(x, W, b, codebook):
    xf = x.reshape(_ROWS, _DIM)
    b2 = b.reshape(1, _G * _N)
    idx, cperp, pperp = _tc_call(xf, W, b2)
    idx3 = idx.reshape(_NW, _NCH, _CH)
    table = codebook.reshape(_G * _N, _VD)
    rows = _sc_gather(table, idx3)
    out = rows.reshape(_B, _T, _G * _VD)
    return out, cperp.reshape(()), pperp.reshape(())
